# Initial kernel scaffold; baseline (speedup 1.0000x reference)
import jax
import jax.numpy as jnp
from jax.experimental import pallas as pl


def kernel(features, edge_index, W1, b1, W2, b2, sW, sb):
    raise NotImplementedError("write your pallas kernel here")


# speedup vs baseline: 4.5830x; 4.5830x over previous
import functools

import numpy as np

import jax
import jax.numpy as jnp
from jax import lax
from jax.experimental import pallas as pl
from jax.experimental.pallas import tpu as pltpu
from jax.experimental.pallas import tpu_sc as plsc

N = 10000
E = 320000
IN_DIM = 128
OUT = 64
K = 12

NC = 2
NS = 16
NW = NC * NS
NPAD = 10240
RPS = NPAD // NS
EPW = E // NW
C = 80
NCHUNK = EPW // C

def _deg_body(dst_hbm, zeros_hbm, ones_hbm, out_hbm, valb, dstb, acc_sh):
    c = lax.axis_index("c")
    s = lax.axis_index("s")
    w = c * jnp.int32(NS) + s
    pltpu.sync_copy(zeros_hbm, valb)
    for j in range(RPS // C):
        pltpu.sync_copy(valb, acc_sh.at[pl.ds(s * jnp.int32(RPS) + jnp.int32(j * C), C)])
    plsc.subcore_barrier()
    pltpu.sync_copy(ones_hbm, valb)

    @pl.loop(jnp.int32(0), jnp.int32(NCHUNK))
    def _chunks(i):
        base = w * jnp.int32(EPW) + i * jnp.int32(C)
        pltpu.sync_copy(dst_hbm.at[pl.ds(base, C)], dstb)
        pltpu.sync_copy(valb, acc_sh.at[dstb], add=True)
    plsc.subcore_barrier()
    for j in range(RPS // C):
        pltpu.sync_copy(acc_sh.at[pl.ds(s * jnp.int32(RPS) + jnp.int32(j * C), C)], valb)
        pltpu.sync_copy(valb, out_hbm.at[pl.ds(c * jnp.int32(NPAD) + s * jnp.int32(RPS) + jnp.int32(j * C), C)])


def _hop_body(x_hbm, src_hbm, dst_hbm, zrows_hbm, out_hbm,
                rows, srcb, dstb, acc_sh, sem):
    c = lax.axis_index("c")
    s = lax.axis_index("s")
    w = c * jnp.int32(NS) + s
    pltpu.sync_copy(zrows_hbm, rows)
    for j in range(RPS // C):
        pltpu.sync_copy(rows, acc_sh.at[pl.ds(s * jnp.int32(RPS) + jnp.int32(j * C), C)])
    plsc.subcore_barrier()

    @pl.loop(jnp.int32(0), jnp.int32(NCHUNK))
    def _chunks(i):
        base = w * jnp.int32(EPW) + i * jnp.int32(C)
        pltpu.sync_copy(src_hbm.at[pl.ds(base, C)], srcb)
        pltpu.sync_copy(dst_hbm.at[pl.ds(base, C)], dstb)
        pltpu.async_copy(x_hbm.at[srcb], rows, sem).wait()
        pltpu.sync_copy(rows, acc_sh.at[dstb], add=True)
    plsc.subcore_barrier()
    for j in range(RPS // C):
        pltpu.sync_copy(acc_sh.at[pl.ds(s * jnp.int32(RPS) + jnp.int32(j * C), C)], rows)
        pltpu.sync_copy(rows, out_hbm.at[pl.ds(c * jnp.int32(NPAD) + s * jnp.int32(RPS) + jnp.int32(j * C), C)])


@functools.lru_cache(maxsize=None)
def _sc_kernels():
    mesh = plsc.VectorSubcoreMesh(
        core_axis_name="c", subcore_axis_name="s",
        num_cores=NC, num_subcores=NS)
    deg_k = pl.kernel(
        _deg_body,
        out_type=jax.ShapeDtypeStruct((NC * NPAD,), jnp.float32),
        mesh=mesh,
        scratch_types=[
            pltpu.VMEM((C,), jnp.float32),
            pltpu.VMEM((C,), jnp.int32),
            pltpu.VMEM_SHARED((NPAD,), jnp.float32),
        ],
    )
    hop_k = pl.kernel(
        _hop_body,
        out_type=jax.ShapeDtypeStruct((NC * NPAD, OUT), jnp.float32),
        mesh=mesh,
        scratch_types=[
            pltpu.VMEM((C, OUT), jnp.float32),
            pltpu.VMEM((C,), jnp.int32),
            pltpu.VMEM((C,), jnp.int32),
            pltpu.VMEM_SHARED((NPAD, OUT), jnp.float32),
            pltpu.SemaphoreType.DMA,
        ],
        compiler_params=pltpu.CompilerParams(use_tc_tiling_on_sc=False),
    )
    return deg_k, hop_k


_BLK = 1024
_Z = np.int32(0)


def _mlp_body(x_ref, w1t_ref, b1_ref, w2t_ref, b2_ref, o_ref):
    h = jnp.maximum(
        jnp.dot(x_ref[...], w1t_ref[...], preferred_element_type=jnp.float32)
        + b1_ref[...], 0.0)
    o_ref[...] = (
        jnp.dot(h, w2t_ref[...], preferred_element_type=jnp.float32)
        + b2_ref[...])


def _mlp(x, w1t, b1, w2t, b2):
    grid = NPAD // _BLK
    return pl.pallas_call(
        _mlp_body,
        grid=(grid,),
        in_specs=[
            pl.BlockSpec((_BLK, IN_DIM), lambda i: (i, _Z)),
            pl.BlockSpec((IN_DIM, OUT), lambda i: (_Z, _Z)),
            pl.BlockSpec((1, OUT), lambda i: (_Z, _Z)),
            pl.BlockSpec((OUT, OUT), lambda i: (_Z, _Z)),
            pl.BlockSpec((1, OUT), lambda i: (_Z, _Z)),
        ],
        out_specs=pl.BlockSpec((_BLK, OUT), lambda i: (i, _Z)),
        out_shape=jax.ShapeDtypeStruct((NPAD, OUT), jnp.float32),
    )(x, w1t, b1, w2t, b2)


def _norm_body(deg_ref, o_ref):
    d = jnp.sum(deg_ref[...], axis=1, keepdims=True)
    o_ref[...] = jnp.where(d > 0.0, lax.rsqrt(jnp.maximum(d, 1.0)), 0.0)


def _norm(degs_t):
    grid = NPAD // _BLK
    return pl.pallas_call(
        _norm_body,
        grid=(grid,),
        in_specs=[pl.BlockSpec((_BLK, NC), lambda i: (i, _Z))],
        out_specs=pl.BlockSpec((_BLK, 1), lambda i: (i, _Z)),
        out_shape=jax.ShapeDtypeStruct((NPAD, 1), jnp.float32),
    )(degs_t)


def _scale0_body(h_ref, n_ref, sw_ref, sb_ref, x0_ref, acc_ref):
    h = h_ref[...]
    x0_ref[...] = h * n_ref[...]
    score = jnp.sum(h * sw_ref[...], axis=1, keepdims=True) + sb_ref[...]
    acc_ref[...] = h * jax.nn.sigmoid(score)


def _scale0(h, norm, sw, sb):
    grid = NPAD // _BLK
    return pl.pallas_call(
        _scale0_body,
        grid=(grid,),
        in_specs=[
            pl.BlockSpec((_BLK, OUT), lambda i: (i, _Z)),
            pl.BlockSpec((_BLK, 1), lambda i: (i, _Z)),
            pl.BlockSpec((1, OUT), lambda i: (_Z, _Z)),
            pl.BlockSpec((1, 1), lambda i: (_Z, _Z)),
        ],
        out_specs=[
            pl.BlockSpec((_BLK, OUT), lambda i: (i, _Z)),
            pl.BlockSpec((_BLK, OUT), lambda i: (i, _Z)),
        ],
        out_shape=[
            jax.ShapeDtypeStruct((NPAD, OUT), jnp.float32),
            jax.ShapeDtypeStruct((NPAD, OUT), jnp.float32),
        ],
    )(h, norm, sw, sb)


def _glue_body(y_ref, n_ref, sw_ref, sb_ref, acc_ref, xk_ref, accout_ref):
    nrm = n_ref[...]
    feats = (y_ref[0] + y_ref[1]) * nrm
    xk_ref[...] = feats * nrm
    score = jnp.sum(feats * sw_ref[...], axis=1, keepdims=True) + sb_ref[...]
    accout_ref[...] = acc_ref[...] + feats * jax.nn.sigmoid(score)


def _glue(y, norm, sw, sb, acc):
    grid = NPAD // _BLK
    return pl.pallas_call(
        _glue_body,
        grid=(grid,),
        in_specs=[
            pl.BlockSpec((NC, _BLK, OUT), lambda i: (_Z, i, _Z)),
            pl.BlockSpec((_BLK, 1), lambda i: (i, _Z)),
            pl.BlockSpec((1, OUT), lambda i: (_Z, _Z)),
            pl.BlockSpec((1, 1), lambda i: (_Z, _Z)),
            pl.BlockSpec((_BLK, OUT), lambda i: (i, _Z)),
        ],
        out_specs=[
            pl.BlockSpec((_BLK, OUT), lambda i: (i, _Z)),
            pl.BlockSpec((_BLK, OUT), lambda i: (i, _Z)),
        ],
        out_shape=[
            jax.ShapeDtypeStruct((NPAD, OUT), jnp.float32),
            jax.ShapeDtypeStruct((NPAD, OUT), jnp.float32),
        ],
    )(y, norm, sw, sb, acc)


def kernel(features, edge_index, W1, b1, W2, b2, sW, sb):
    src = edge_index[0].astype(jnp.int32)
    dst = edge_index[1].astype(jnp.int32)

    fpad = jnp.zeros((NPAD, IN_DIM), jnp.float32).at[:N].set(features)
    w1t = W1.T
    w2t = W2.T
    b1r = b1.reshape(1, OUT)
    b2r = b2.reshape(1, OUT)
    swr = sW.reshape(1, OUT)
    sbr = sb.reshape(1, 1)

    zeros_c = jnp.zeros((C,), jnp.float32)
    ones_c = jnp.ones((C,), jnp.float32)
    zrows = jnp.zeros((C, OUT), jnp.float32)

    deg_k, hop_k = _sc_kernels()
    h = _mlp(fpad, w1t, b1r, w2t, b2r)
    degs = deg_k(dst, zeros_c, ones_c)
    norm = _norm(degs.reshape(NC, NPAD).T)
    x, acc = _scale0(h, norm, swr, sbr)
    for _ in range(K):
        y = hop_k(x, src, dst, zrows)
        x, acc = _glue(y.reshape(NC, NPAD, OUT), norm, swr, sbr, acc)
    return acc[:N]


# speedup vs baseline: 5.0308x; 1.0977x over previous
import functools

import numpy as np

import jax
import jax.numpy as jnp
from jax import lax
from jax.experimental import pallas as pl
from jax.experimental.pallas import tpu as pltpu
from jax.experimental.pallas import tpu_sc as plsc

N = 10000
E = 320000
IN_DIM = 128
OUT = 64
K = 12

NC = 2
NS = 16
NW = NC * NS
NPAD = 10240
RPS = NPAD // NS
EPW = E // NW
C = 80
NCHUNK = EPW // C
HC = 128
HCHUNK = 80
EPWP = HC * HCHUNK
EPAD = NW * EPWP

def _deg_body(dst_hbm, zeros_hbm, ones_hbm, out_hbm, valb, dstb, acc_sh):
    c = lax.axis_index("c")
    s = lax.axis_index("s")
    w = c * jnp.int32(NS) + s
    pltpu.sync_copy(zeros_hbm, valb)
    for j in range(RPS // C):
        pltpu.sync_copy(valb, acc_sh.at[pl.ds(s * jnp.int32(RPS) + jnp.int32(j * C), C)])
    plsc.subcore_barrier()
    pltpu.sync_copy(ones_hbm, valb)

    @pl.loop(jnp.int32(0), jnp.int32(NCHUNK))
    def _chunks(i):
        base = w * jnp.int32(EPW) + i * jnp.int32(C)
        pltpu.sync_copy(dst_hbm.at[pl.ds(base, C)], dstb)
        pltpu.sync_copy(valb, acc_sh.at[dstb], add=True)
    plsc.subcore_barrier()
    for j in range(RPS // C):
        pltpu.sync_copy(acc_sh.at[pl.ds(s * jnp.int32(RPS) + jnp.int32(j * C), C)], valb)
        pltpu.sync_copy(valb, out_hbm.at[pl.ds(c * jnp.int32(NPAD) + s * jnp.int32(RPS) + jnp.int32(j * C), C)])


def _hop_body(x_hbm, src_hbm, dst_hbm, zrows_hbm, out_hbm,
              srcall, dstall, buf0, buf1, acc_sh, sem0, sem1):
    c = lax.axis_index("c")
    s = lax.axis_index("s")
    w = c * jnp.int32(NS) + s
    pltpu.sync_copy(src_hbm.at[w], srcall)
    pltpu.sync_copy(dst_hbm.at[w], dstall)
    pltpu.sync_copy(zrows_hbm, buf0)
    for j in range(RPS // HC):
        pltpu.sync_copy(buf0, acc_sh.at[pl.ds(s * jnp.int32(RPS) + jnp.int32(j * HC), HC)])
    plsc.subcore_barrier()

    pltpu.async_copy(x_hbm.at[srcall.at[jnp.int32(0)]], buf0, sem0)

    @pl.loop(jnp.int32(0), jnp.int32(HCHUNK // 2))
    def _pairs(i):
        i2 = i * jnp.int32(2)
        pltpu.async_copy(x_hbm.at[srcall.at[i2 + jnp.int32(1)]], buf1, sem1)
        pltpu.make_async_copy(x_hbm.at[pl.ds(jnp.int32(0), HC)], buf0, sem0).wait()
        pltpu.sync_copy(buf0, acc_sh.at[dstall.at[i2]], add=True)
        pltpu.async_copy(
            x_hbm.at[srcall.at[jnp.minimum(i2 + jnp.int32(2), jnp.int32(HCHUNK - 1))]],
            buf0, sem0)
        pltpu.make_async_copy(x_hbm.at[pl.ds(jnp.int32(0), HC)], buf1, sem1).wait()
        pltpu.sync_copy(buf1, acc_sh.at[dstall.at[i2 + jnp.int32(1)]], add=True)

    pltpu.make_async_copy(x_hbm.at[pl.ds(jnp.int32(0), HC)], buf0, sem0).wait()
    plsc.subcore_barrier()
    for j in range(RPS // HC):
        pltpu.sync_copy(acc_sh.at[pl.ds(s * jnp.int32(RPS) + jnp.int32(j * HC), HC)], buf0)
        pltpu.sync_copy(buf0, out_hbm.at[pl.ds(c * jnp.int32(NPAD) + s * jnp.int32(RPS) + jnp.int32(j * HC), HC)])


@functools.lru_cache(maxsize=None)
def _sc_kernels():
    mesh = plsc.VectorSubcoreMesh(
        core_axis_name="c", subcore_axis_name="s",
        num_cores=NC, num_subcores=NS)
    deg_k = pl.kernel(
        _deg_body,
        out_type=jax.ShapeDtypeStruct((NC * NPAD,), jnp.float32),
        mesh=mesh,
        scratch_types=[
            pltpu.VMEM((C,), jnp.float32),
            pltpu.VMEM((C,), jnp.int32),
            pltpu.VMEM_SHARED((NPAD,), jnp.float32),
        ],
    )
    hop_k = pl.kernel(
        _hop_body,
        out_type=jax.ShapeDtypeStruct((NC * NPAD, OUT), jnp.float32),
        mesh=mesh,
        scratch_types=[
            pltpu.VMEM((HCHUNK, HC), jnp.int32),
            pltpu.VMEM((HCHUNK, HC), jnp.int32),
            pltpu.VMEM((HC, OUT), jnp.float32),
            pltpu.VMEM((HC, OUT), jnp.float32),
            pltpu.VMEM_SHARED((NPAD, OUT), jnp.float32),
            pltpu.SemaphoreType.DMA,
            pltpu.SemaphoreType.DMA,
        ],
        compiler_params=pltpu.CompilerParams(use_tc_tiling_on_sc=False),
    )
    return deg_k, hop_k


_BLK = 1024
_Z = np.int32(0)


def _mlp_body(x_ref, w1t_ref, b1_ref, w2t_ref, b2_ref, o_ref):
    h = jnp.maximum(
        jnp.dot(x_ref[...], w1t_ref[...], preferred_element_type=jnp.float32)
        + b1_ref[...], 0.0)
    o_ref[...] = (
        jnp.dot(h, w2t_ref[...], preferred_element_type=jnp.float32)
        + b2_ref[...])


def _mlp(x, w1t, b1, w2t, b2):
    grid = NPAD // _BLK
    return pl.pallas_call(
        _mlp_body,
        grid=(grid,),
        in_specs=[
            pl.BlockSpec((_BLK, IN_DIM), lambda i: (i, _Z)),
            pl.BlockSpec((IN_DIM, OUT), lambda i: (_Z, _Z)),
            pl.BlockSpec((1, OUT), lambda i: (_Z, _Z)),
            pl.BlockSpec((OUT, OUT), lambda i: (_Z, _Z)),
            pl.BlockSpec((1, OUT), lambda i: (_Z, _Z)),
        ],
        out_specs=pl.BlockSpec((_BLK, OUT), lambda i: (i, _Z)),
        out_shape=jax.ShapeDtypeStruct((NPAD, OUT), jnp.float32),
    )(x, w1t, b1, w2t, b2)


def _norm_body(deg_ref, o_ref):
    d = jnp.sum(deg_ref[...], axis=1, keepdims=True)
    o_ref[...] = jnp.where(d > 0.0, lax.rsqrt(jnp.maximum(d, 1.0)), 0.0)


def _norm(degs_t):
    grid = NPAD // _BLK
    return pl.pallas_call(
        _norm_body,
        grid=(grid,),
        in_specs=[pl.BlockSpec((_BLK, NC), lambda i: (i, _Z))],
        out_specs=pl.BlockSpec((_BLK, 1), lambda i: (i, _Z)),
        out_shape=jax.ShapeDtypeStruct((NPAD, 1), jnp.float32),
    )(degs_t)


def _scale0_body(h_ref, n_ref, sw_ref, sb_ref, x0_ref, acc_ref):
    h = h_ref[...]
    x0_ref[...] = h * n_ref[...]
    score = jnp.sum(h * sw_ref[...], axis=1, keepdims=True) + sb_ref[...]
    acc_ref[...] = h * jax.nn.sigmoid(score)


def _scale0(h, norm, sw, sb):
    grid = NPAD // _BLK
    return pl.pallas_call(
        _scale0_body,
        grid=(grid,),
        in_specs=[
            pl.BlockSpec((_BLK, OUT), lambda i: (i, _Z)),
            pl.BlockSpec((_BLK, 1), lambda i: (i, _Z)),
            pl.BlockSpec((1, OUT), lambda i: (_Z, _Z)),
            pl.BlockSpec((1, 1), lambda i: (_Z, _Z)),
        ],
        out_specs=[
            pl.BlockSpec((_BLK, OUT), lambda i: (i, _Z)),
            pl.BlockSpec((_BLK, OUT), lambda i: (i, _Z)),
        ],
        out_shape=[
            jax.ShapeDtypeStruct((NPAD, OUT), jnp.float32),
            jax.ShapeDtypeStruct((NPAD, OUT), jnp.float32),
        ],
    )(h, norm, sw, sb)


def _glue_body(y_ref, n_ref, sw_ref, sb_ref, acc_ref, xk_ref, accout_ref):
    nrm = n_ref[...]
    feats = (y_ref[0] + y_ref[1]) * nrm
    xk_ref[...] = feats * nrm
    score = jnp.sum(feats * sw_ref[...], axis=1, keepdims=True) + sb_ref[...]
    accout_ref[...] = acc_ref[...] + feats * jax.nn.sigmoid(score)


def _glue(y, norm, sw, sb, acc):
    grid = NPAD // _BLK
    return pl.pallas_call(
        _glue_body,
        grid=(grid,),
        in_specs=[
            pl.BlockSpec((NC, _BLK, OUT), lambda i: (_Z, i, _Z)),
            pl.BlockSpec((_BLK, 1), lambda i: (i, _Z)),
            pl.BlockSpec((1, OUT), lambda i: (_Z, _Z)),
            pl.BlockSpec((1, 1), lambda i: (_Z, _Z)),
            pl.BlockSpec((_BLK, OUT), lambda i: (i, _Z)),
        ],
        out_specs=[
            pl.BlockSpec((_BLK, OUT), lambda i: (i, _Z)),
            pl.BlockSpec((_BLK, OUT), lambda i: (i, _Z)),
        ],
        out_shape=[
            jax.ShapeDtypeStruct((NPAD, OUT), jnp.float32),
            jax.ShapeDtypeStruct((NPAD, OUT), jnp.float32),
        ],
    )(y, norm, sw, sb, acc)


def kernel(features, edge_index, W1, b1, W2, b2, sW, sb):
    src = edge_index[0].astype(jnp.int32)
    dst = edge_index[1].astype(jnp.int32)

    fpad = jnp.zeros((NPAD, IN_DIM), jnp.float32).at[:N].set(features)
    w1t = W1.T
    w2t = W2.T
    b1r = b1.reshape(1, OUT)
    b2r = b2.reshape(1, OUT)
    swr = sW.reshape(1, OUT)
    sbr = sb.reshape(1, 1)

    zeros_c = jnp.zeros((C,), jnp.float32)
    ones_c = jnp.ones((C,), jnp.float32)
    zrows = jnp.zeros((HC, OUT), jnp.float32)

    src3 = jnp.zeros((EPAD,), jnp.int32).at[:E].set(src).reshape(NW, HCHUNK, HC)
    dst3 = jnp.full((EPAD,), NPAD - 1, jnp.int32).at[:E].set(dst).reshape(
        NW, HCHUNK, HC)

    deg_k, hop_k = _sc_kernels()
    h = _mlp(fpad, w1t, b1r, w2t, b2r)
    degs = deg_k(dst, zeros_c, ones_c)
    norm = _norm(degs.reshape(NC, NPAD).T)
    x, acc = _scale0(h, norm, swr, sbr)
    for _ in range(K):
        y = hop_k(x, src3, dst3, zrows)
        x, acc = _glue(y.reshape(NC, NPAD, OUT), norm, swr, sbr, acc)
    return acc[:N]


# speedup vs baseline: 5.1177x; 1.0173x over previous
import functools

import numpy as np

import jax
import jax.numpy as jnp
from jax import lax
from jax.experimental import pallas as pl
from jax.experimental.pallas import tpu as pltpu
from jax.experimental.pallas import tpu_sc as plsc

N = 10000
E = 320000
IN_DIM = 128
OUT = 64
K = 12

NC = 2
NS = 16
NW = NC * NS
NPAD = 10240
RPS = NPAD // NS
EPW = E // NW
C = 80
NCHUNK = EPW // C
HC = 128
HCHUNK = 80
EPWP = HC * HCHUNK
EPAD = NW * EPWP

def _deg_body(dst_hbm, zeros_hbm, ones_hbm, out_hbm, valb, dstb, acc_sh):
    c = lax.axis_index("c")
    s = lax.axis_index("s")
    w = c * jnp.int32(NS) + s
    pltpu.sync_copy(zeros_hbm, valb)
    for j in range(RPS // C):
        pltpu.sync_copy(valb, acc_sh.at[pl.ds(s * jnp.int32(RPS) + jnp.int32(j * C), C)])
    plsc.subcore_barrier()
    pltpu.sync_copy(ones_hbm, valb)

    @pl.loop(jnp.int32(0), jnp.int32(NCHUNK))
    def _chunks(i):
        base = w * jnp.int32(EPW) + i * jnp.int32(C)
        pltpu.sync_copy(dst_hbm.at[pl.ds(base, C)], dstb)
        pltpu.sync_copy(valb, acc_sh.at[dstb], add=True)
    plsc.subcore_barrier()
    for j in range(RPS // C):
        pltpu.sync_copy(acc_sh.at[pl.ds(s * jnp.int32(RPS) + jnp.int32(j * C), C)], valb)
        pltpu.sync_copy(valb, out_hbm.at[pl.ds(c * jnp.int32(NPAD) + s * jnp.int32(RPS) + jnp.int32(j * C), C)])


def _hop_body(x_hbm, src_hbm, dst_hbm, zrows_hbm, out_hbm,
              srcall, dstall, b0, b1, b2, b3, acc_sh,
              g0, g1, g2, g3, s0, s1, s2, s3):
    c = lax.axis_index("c")
    s = lax.axis_index("s")
    w = c * jnp.int32(NS) + s
    bufs = (b0, b1, b2, b3)
    gsems = (g0, g1, g2, g3)
    ssems = (s0, s1, s2, s3)

    def gather(ch, q):
        pltpu.async_copy(x_hbm.at[srcall.at[ch]], bufs[q], gsems[q])

    def wait_g(q):
        pltpu.make_async_copy(
            x_hbm.at[pl.ds(jnp.int32(0), HC)], bufs[q], gsems[q]).wait()

    def scat(ch, q):
        pltpu.async_copy(bufs[q], acc_sh.at[dstall.at[ch]], ssems[q],
                         add=True)

    def wait_s(q):
        pltpu.make_async_copy(
            bufs[q], acc_sh.at[pl.ds(jnp.int32(0), HC)], ssems[q]).wait()

    pltpu.sync_copy(src_hbm.at[w], srcall)
    pltpu.sync_copy(dst_hbm.at[w], dstall)
    pltpu.sync_copy(zrows_hbm, b0)
    for j in range(RPS // HC):
        pltpu.sync_copy(b0, acc_sh.at[pl.ds(s * jnp.int32(RPS) + jnp.int32(j * HC), HC)])
    plsc.subcore_barrier()

    Z = jnp.int32(0)
    gather(Z, 0)
    gather(jnp.int32(1), 1)
    gather(jnp.int32(2), 2); wait_g(0); scat(Z, 0)
    gather(jnp.int32(3), 3); wait_g(1); scat(jnp.int32(1), 1)
    wait_s(0); gather(jnp.int32(4), 0); wait_g(2); scat(jnp.int32(2), 2)
    wait_s(1); gather(jnp.int32(5), 1); wait_g(3); scat(jnp.int32(3), 3)

    @pl.loop(jnp.int32(1), jnp.int32(HCHUNK // 4 - 1))
    def _groups(i):
        i4 = i * jnp.int32(4)
        wait_s(2); gather(i4 + jnp.int32(2), 2); wait_g(0); scat(i4, 0)
        wait_s(3); gather(i4 + jnp.int32(3), 3); wait_g(1); scat(i4 + jnp.int32(1), 1)
        wait_s(0); gather(i4 + jnp.int32(4), 0); wait_g(2); scat(i4 + jnp.int32(2), 2)
        wait_s(1); gather(i4 + jnp.int32(5), 1); wait_g(3); scat(i4 + jnp.int32(3), 3)

    L = jnp.int32(HCHUNK - 4)
    wait_s(2); gather(L + jnp.int32(2), 2); wait_g(0); scat(L, 0)
    wait_s(3); gather(L + jnp.int32(3), 3); wait_g(1); scat(L + jnp.int32(1), 1)
    wait_s(0); wait_g(2); scat(L + jnp.int32(2), 2)
    wait_s(1); wait_g(3); scat(L + jnp.int32(3), 3)
    wait_s(2); wait_s(3)

    plsc.subcore_barrier()
    for j in range(RPS // HC):
        pltpu.sync_copy(acc_sh.at[pl.ds(s * jnp.int32(RPS) + jnp.int32(j * HC), HC)], b0)
        pltpu.sync_copy(b0, out_hbm.at[pl.ds(c * jnp.int32(NPAD) + s * jnp.int32(RPS) + jnp.int32(j * HC), HC)])


@functools.lru_cache(maxsize=None)
def _sc_kernels():
    mesh = plsc.VectorSubcoreMesh(
        core_axis_name="c", subcore_axis_name="s",
        num_cores=NC, num_subcores=NS)
    deg_k = pl.kernel(
        _deg_body,
        out_type=jax.ShapeDtypeStruct((NC * NPAD,), jnp.float32),
        mesh=mesh,
        scratch_types=[
            pltpu.VMEM((C,), jnp.float32),
            pltpu.VMEM((C,), jnp.int32),
            pltpu.VMEM_SHARED((NPAD,), jnp.float32),
        ],
    )
    hop_k = pl.kernel(
        _hop_body,
        out_type=jax.ShapeDtypeStruct((NC * NPAD, OUT), jnp.float32),
        mesh=mesh,
        scratch_types=[
            pltpu.VMEM((HCHUNK, HC), jnp.int32),
            pltpu.VMEM((HCHUNK, HC), jnp.int32),
            pltpu.VMEM((HC, OUT), jnp.float32),
            pltpu.VMEM((HC, OUT), jnp.float32),
            pltpu.VMEM((HC, OUT), jnp.float32),
            pltpu.VMEM((HC, OUT), jnp.float32),
            pltpu.VMEM_SHARED((NPAD, OUT), jnp.float32),
        ] + [pltpu.SemaphoreType.DMA] * 8,
        compiler_params=pltpu.CompilerParams(use_tc_tiling_on_sc=False),
    )
    return deg_k, hop_k


_BLK = 1024
_Z = np.int32(0)


def _mlp_body(x_ref, w1t_ref, b1_ref, w2t_ref, b2_ref, o_ref):
    h = jnp.maximum(
        jnp.dot(x_ref[...], w1t_ref[...], preferred_element_type=jnp.float32)
        + b1_ref[...], 0.0)
    o_ref[...] = (
        jnp.dot(h, w2t_ref[...], preferred_element_type=jnp.float32)
        + b2_ref[...])


def _mlp(x, w1t, b1, w2t, b2):
    grid = NPAD // _BLK
    return pl.pallas_call(
        _mlp_body,
        grid=(grid,),
        in_specs=[
            pl.BlockSpec((_BLK, IN_DIM), lambda i: (i, _Z)),
            pl.BlockSpec((IN_DIM, OUT), lambda i: (_Z, _Z)),
            pl.BlockSpec((1, OUT), lambda i: (_Z, _Z)),
            pl.BlockSpec((OUT, OUT), lambda i: (_Z, _Z)),
            pl.BlockSpec((1, OUT), lambda i: (_Z, _Z)),
        ],
        out_specs=pl.BlockSpec((_BLK, OUT), lambda i: (i, _Z)),
        out_shape=jax.ShapeDtypeStruct((NPAD, OUT), jnp.float32),
    )(x, w1t, b1, w2t, b2)


def _norm_body(deg_ref, o_ref):
    d = jnp.sum(deg_ref[...], axis=1, keepdims=True)
    o_ref[...] = jnp.where(d > 0.0, lax.rsqrt(jnp.maximum(d, 1.0)), 0.0)


def _norm(degs_t):
    grid = NPAD // _BLK
    return pl.pallas_call(
        _norm_body,
        grid=(grid,),
        in_specs=[pl.BlockSpec((_BLK, NC), lambda i: (i, _Z))],
        out_specs=pl.BlockSpec((_BLK, 1), lambda i: (i, _Z)),
        out_shape=jax.ShapeDtypeStruct((NPAD, 1), jnp.float32),
    )(degs_t)


def _scale0_body(h_ref, n_ref, sw_ref, sb_ref, x0_ref, acc_ref):
    h = h_ref[...]
    x0_ref[...] = h * n_ref[...]
    score = jnp.sum(h * sw_ref[...], axis=1, keepdims=True) + sb_ref[...]
    acc_ref[...] = h * jax.nn.sigmoid(score)


def _scale0(h, norm, sw, sb):
    grid = NPAD // _BLK
    return pl.pallas_call(
        _scale0_body,
        grid=(grid,),
        in_specs=[
            pl.BlockSpec((_BLK, OUT), lambda i: (i, _Z)),
            pl.BlockSpec((_BLK, 1), lambda i: (i, _Z)),
            pl.BlockSpec((1, OUT), lambda i: (_Z, _Z)),
            pl.BlockSpec((1, 1), lambda i: (_Z, _Z)),
        ],
        out_specs=[
            pl.BlockSpec((_BLK, OUT), lambda i: (i, _Z)),
            pl.BlockSpec((_BLK, OUT), lambda i: (i, _Z)),
        ],
        out_shape=[
            jax.ShapeDtypeStruct((NPAD, OUT), jnp.float32),
            jax.ShapeDtypeStruct((NPAD, OUT), jnp.float32),
        ],
    )(h, norm, sw, sb)


def _glue_body(y_ref, n_ref, sw_ref, sb_ref, acc_ref, xk_ref, accout_ref):
    nrm = n_ref[...]
    feats = (y_ref[0] + y_ref[1]) * nrm
    xk_ref[...] = feats * nrm
    score = jnp.sum(feats * sw_ref[...], axis=1, keepdims=True) + sb_ref[...]
    accout_ref[...] = acc_ref[...] + feats * jax.nn.sigmoid(score)


def _glue(y, norm, sw, sb, acc):
    grid = NPAD // _BLK
    return pl.pallas_call(
        _glue_body,
        grid=(grid,),
        in_specs=[
            pl.BlockSpec((NC, _BLK, OUT), lambda i: (_Z, i, _Z)),
            pl.BlockSpec((_BLK, 1), lambda i: (i, _Z)),
            pl.BlockSpec((1, OUT), lambda i: (_Z, _Z)),
            pl.BlockSpec((1, 1), lambda i: (_Z, _Z)),
            pl.BlockSpec((_BLK, OUT), lambda i: (i, _Z)),
        ],
        out_specs=[
            pl.BlockSpec((_BLK, OUT), lambda i: (i, _Z)),
            pl.BlockSpec((_BLK, OUT), lambda i: (i, _Z)),
        ],
        out_shape=[
            jax.ShapeDtypeStruct((NPAD, OUT), jnp.float32),
            jax.ShapeDtypeStruct((NPAD, OUT), jnp.float32),
        ],
    )(y, norm, sw, sb, acc)


def kernel(features, edge_index, W1, b1, W2, b2, sW, sb):
    src = edge_index[0].astype(jnp.int32)
    dst = edge_index[1].astype(jnp.int32)

    fpad = jnp.zeros((NPAD, IN_DIM), jnp.float32).at[:N].set(features)
    w1t = W1.T
    w2t = W2.T
    b1r = b1.reshape(1, OUT)
    b2r = b2.reshape(1, OUT)
    swr = sW.reshape(1, OUT)
    sbr = sb.reshape(1, 1)

    zeros_c = jnp.zeros((C,), jnp.float32)
    ones_c = jnp.ones((C,), jnp.float32)
    zrows = jnp.zeros((HC, OUT), jnp.float32)

    src3 = jnp.zeros((EPAD,), jnp.int32).at[:E].set(src).reshape(NW, HCHUNK, HC)
    dst3 = jnp.full((EPAD,), NPAD - 1, jnp.int32).at[:E].set(dst).reshape(
        NW, HCHUNK, HC)

    deg_k, hop_k = _sc_kernels()
    h = _mlp(fpad, w1t, b1r, w2t, b2r)
    degs = deg_k(dst, zeros_c, ones_c)
    norm = _norm(degs.reshape(NC, NPAD).T)
    x, acc = _scale0(h, norm, swr, sbr)
    for _ in range(K):
        y = hop_k(x, src3, dst3, zrows)
        x, acc = _glue(y.reshape(NC, NPAD, OUT), norm, swr, sbr, acc)
    return acc[:N]
